# baseline (speedup 1.0000x reference)
import functools

import jax
import jax.numpy as jnp
from jax import lax
from jax.experimental import pallas as pl
from jax.experimental.pallas import tpu as pltpu
from jax.experimental.pallas import tpu_sc as plsc

NC, NS, LANES = 2, 16, 16
NW = NC * NS
CHUNK = 64
DEGZ_ROWS = 128


def _zero_rows(ref, nrows, width):
    def body(i, _):
        for j in range(width // LANES):
            ref[i, pl.ds(j * LANES, LANES)] = jnp.zeros((LANES,), jnp.float32)
        return 0

    lax.fori_loop(0, nrows, body, 0, unroll=False)


NBUF = 4


def _segsum_body(compute_deg, feat, rows_per_w, agg_rows,
                 h, idxm, *rest):
    if compute_deg:
        (out_agg, out_deg, agg_sp, deg_sp, rows_v, idx_v, ones_v, degz,
         sem_i, sem_g, sem_s, sem_d) = rest
    else:
        out_agg, agg_sp, rows_v, idx_v, sem_i, sem_g, sem_s = rest
        deg_sp = ones_v = degz = sem_d = None

    c = lax.axis_index("c")
    s = lax.axis_index("s")
    wid = s * NC + c

    zrows = agg_rows // NS
    _zero_rows(rows_v.at[0], CHUNK, feat)
    base_z = s * zrows
    off = 0
    while off < zrows:
        sz = min(CHUNK, zrows - off)
        pltpu.sync_copy(rows_v.at[0, pl.ds(0, sz)], agg_sp.at[pl.ds(base_z + off, sz)])
        off += sz
    if compute_deg:
        _zero_rows(degz, DEGZ_ROWS, LANES)
        off = 0
        while off < zrows:
            sz = min(DEGZ_ROWS, zrows - off)
            pltpu.sync_copy(degz.at[pl.ds(0, sz)], deg_sp.at[pl.ds(base_z + off, sz)])
            off += sz

        def fill_ones(i, _):
            ones_v[i, :] = jnp.ones((LANES,), jnp.float32)
            return 0

        lax.fori_loop(0, CHUNK, fill_ones, 0, unroll=False)
    plsc.subcore_barrier()

    base_row = wid * rows_per_w

    def fire_idx(t, b):
        pltpu.async_copy(idxm.at[base_row + t], idx_v.at[b], sem_i.at[b])

    def wait_idx(b):
        pltpu.make_async_copy(idxm.at[base_row], idx_v.at[b], sem_i.at[b]).wait()

    def fire_gather(b):
        pltpu.async_copy(h.at[idx_v.at[b, 0]], rows_v.at[b], sem_g.at[b])

    def wait_gather(b):
        pltpu.make_async_copy(h.at[idx_v.at[b, 0]], rows_v.at[b], sem_g.at[b]).wait()

    def fire_scatter(b):
        pltpu.async_copy(rows_v.at[b], agg_sp.at[idx_v.at[b, 1]], sem_s.at[b],
                         add=True)
        if compute_deg:
            pltpu.async_copy(ones_v, deg_sp.at[idx_v.at[b, 1]], sem_d.at[b],
                             add=True)

    def wait_scatter(b):
        pltpu.make_async_copy(rows_v.at[b], agg_sp.at[idx_v.at[b, 1]],
                              sem_s.at[b]).wait()
        if compute_deg:
            pltpu.make_async_copy(ones_v, deg_sp.at[idx_v.at[b, 1]],
                                  sem_d.at[b]).wait()

    fire_idx(0, 0)
    fire_idx(1, 1)
    wait_idx(0)
    fire_gather(0)
    fire_idx(2, 2)
    wait_idx(1)
    fire_gather(1)
    wait_gather(0)
    fire_scatter(0)
    fire_idx(3, 3)
    wait_idx(2)
    fire_gather(2)
    wait_gather(1)
    fire_scatter(1)

    def step(g, _):
        for b in range(NBUF):
            t = g * NBUF + b
            wait_scatter(b)
            fire_idx(t, b)
            wait_idx((b + 3) % NBUF)
            fire_gather((b + 3) % NBUF)
            wait_gather((b + 2) % NBUF)
            fire_scatter((b + 2) % NBUF)
        return 0

    lax.fori_loop(1, rows_per_w // NBUF, step, 0, unroll=False)

    last = rows_per_w - 1
    wait_idx(3)
    fire_gather(3)
    wait_gather(2)
    fire_scatter(2)
    wait_gather(3)
    fire_scatter(3)
    for b in range(NBUF):
        wait_scatter(b)
    plsc.subcore_barrier()

    orows = agg_rows // NS
    base_o = s * orows
    off = 0
    while off < orows:
        sz = min(CHUNK, orows - off)
        pltpu.sync_copy(agg_sp.at[pl.ds(base_o + off, sz)], rows_v.at[0, pl.ds(0, sz)])
        pltpu.sync_copy(rows_v.at[0, pl.ds(0, sz)], out_agg.at[c, pl.ds(base_o + off, sz)])
        off += sz
    if compute_deg:
        off = 0
        while off < orows:
            sz = min(DEGZ_ROWS, orows - off)
            pltpu.sync_copy(deg_sp.at[pl.ds(base_o + off, sz)], degz.at[pl.ds(0, sz)])
            pltpu.sync_copy(degz.at[pl.ds(0, sz)], out_deg.at[c, pl.ds(base_o + off, sz)])
            off += sz


def _segsum(h, idxm, compute_deg):
    n_nodes, feat = h.shape
    erows = idxm.shape[0]
    rows_per_w = erows // NW
    agg_rows = -(-(n_nodes + 1) // (NS * 8)) * (NS * 8)
    assert erows % NW == 0

    mesh = plsc.VectorSubcoreMesh(
        core_axis_name="c", subcore_axis_name="s", num_cores=NC, num_subcores=NS
    )
    out_type = [jax.ShapeDtypeStruct((NC, agg_rows, feat), jnp.float32)]
    if compute_deg:
        out_type.append(jax.ShapeDtypeStruct((NC, agg_rows, LANES), jnp.float32))
        scratch = [
            pltpu.VMEM_SHARED((agg_rows, feat), jnp.float32),
            pltpu.VMEM_SHARED((agg_rows, LANES), jnp.float32),
            pltpu.VMEM((NBUF, CHUNK, feat), jnp.float32),
            pltpu.VMEM((NBUF, 2, CHUNK), jnp.int32),
            pltpu.VMEM((CHUNK, LANES), jnp.float32),
            pltpu.VMEM((DEGZ_ROWS, LANES), jnp.float32),
            pltpu.SemaphoreType.DMA((NBUF,)),
            pltpu.SemaphoreType.DMA((NBUF,)),
            pltpu.SemaphoreType.DMA((NBUF,)),
            pltpu.SemaphoreType.DMA((NBUF,)),
        ]
    else:
        scratch = [
            pltpu.VMEM_SHARED((agg_rows, feat), jnp.float32),
            pltpu.VMEM((NBUF, CHUNK, feat), jnp.float32),
            pltpu.VMEM((NBUF, 2, CHUNK), jnp.int32),
            pltpu.SemaphoreType.DMA((NBUF,)),
            pltpu.SemaphoreType.DMA((NBUF,)),
            pltpu.SemaphoreType.DMA((NBUF,)),
        ]

    body = functools.partial(_segsum_body, compute_deg, feat,
                             rows_per_w, agg_rows)
    fn = pl.kernel(body, out_type=tuple(out_type), mesh=mesh,
                   scratch_types=tuple(scratch),
                   compiler_params=pltpu.CompilerParams(use_tc_tiling_on_sc=False))
    return fn(h, idxm)


def _sage_block(h, aggp, degp, wl, bl, wr, g, b):
    n = h.shape[0]
    dp = degp[...]
    deg = dp[0, :n, 0:1] + dp[1, :n, 0:1]
    inv = 1.0 / jnp.maximum(deg, 1.0)
    mean = (aggp[0, :n] + aggp[1, :n]) * inv
    lin = (
        jnp.dot(mean, wl[...], preferred_element_type=jnp.float32)
        + bl[...]
        + jnp.dot(h[...], wr[...], preferred_element_type=jnp.float32)
    )
    mu = jnp.mean(lin, axis=0, keepdims=True)
    xc = lin - mu
    var = jnp.mean(xc * xc, axis=0, keepdims=True)
    y = g[...] * xc * lax.rsqrt(var + 1e-5) + b[...]
    return jnp.maximum(y, 0.0)


def _layer_mid_body(h, aggp, degp, wl, bl, wr, g, b, out):
    out[...] = _sage_block(h, aggp, degp, wl, bl, wr, g, b)


def _layer_final_body(h, aggp, degp, wl, bl, wr, g, b, wc, bc, out):
    hr = _sage_block(h, aggp, degp, wl, bl, wr, g, b)
    logits = jnp.dot(hr, wc[...], preferred_element_type=jnp.float32) + bc[...]
    m = jnp.max(logits, axis=1, keepdims=True)
    z = logits - m
    lse = jnp.log(jnp.sum(jnp.exp(z), axis=1, keepdims=True))
    out[...] = z - lse


def _layer_mid(h, aggp, degp, wl, bl, wr, g, b):
    n, feat = h.shape
    return pl.pallas_call(
        _layer_mid_body,
        out_shape=jax.ShapeDtypeStruct((n, feat), jnp.float32),
    )(h, aggp, degp, wl, bl.reshape(1, -1), wr, g.reshape(1, -1), b.reshape(1, -1))


def _layer_final(h, aggp, degp, wl, bl, wr, g, b, wc, bc):
    n = h.shape[0]
    ncls = wc.shape[1]
    return pl.pallas_call(
        _layer_final_body,
        out_shape=jax.ShapeDtypeStruct((n, ncls), jnp.float32),
    )(h, aggp, degp, wl, bl.reshape(1, -1), wr, g.reshape(1, -1), b.reshape(1, -1),
      wc, bc.reshape(1, -1))


def kernel(x, edge_index, Wl0, bl0, Wr0, gamma0, beta0, Wl1, bl1, Wr1, gamma1, beta1, Wl2, bl2, Wr2, gamma2, beta2, Wc, bc):
    n_nodes = x.shape[0]
    e = edge_index.shape[1]
    epad = -(-e // (CHUNK * NW * NBUF)) * (CHUNK * NW * NBUF)
    src = edge_index[0]
    dst = edge_index[1]
    if epad > e:
        pad = epad - e
        src = jnp.concatenate([src, jnp.zeros((pad,), jnp.int32)])
        dst = jnp.concatenate([dst, jnp.full((pad,), n_nodes, jnp.int32)])
    idxm = jnp.stack(
        [src.reshape(epad // CHUNK, CHUNK), dst.reshape(epad // CHUNK, CHUNK)],
        axis=1,
    )

    agg0, degp = _segsum(x, idxm, compute_deg=True)
    h1 = _layer_mid(x, agg0, degp, Wl0, bl0, Wr0, gamma0, beta0)
    (agg1,) = _segsum(h1, idxm, compute_deg=False)
    h2 = _layer_mid(h1, agg1, degp, Wl1, bl1, Wr1, gamma1, beta1)
    (agg2,) = _segsum(h2, idxm, compute_deg=False)
    return _layer_final(h2, agg2, degp, Wl2, bl2, Wr2, gamma2, beta2, Wc, bc)


# speedup vs baseline: 2.5405x; 2.5405x over previous
import functools

import jax
import jax.numpy as jnp
from jax import lax
from jax.experimental import pallas as pl
from jax.experimental.pallas import tpu as pltpu
from jax.experimental.pallas import tpu_sc as plsc

NC, NS, LANES = 2, 16, 16
HALF = 64
CHUNK = 128
NBUF = 4
DEGZ_ROWS = 128


def _zero_rows(ref, nrows, width):
    def body(i, _):
        for j in range(width // LANES):
            ref[i, pl.ds(j * LANES, LANES)] = jnp.zeros((LANES,), jnp.float32)
        return 0

    lax.fori_loop(0, nrows, body, 0, unroll=False)


def _segsum_body(compute_deg, rows_per_tile, agg_rows, hs, idxm, *rest):
    if compute_deg:
        (out_agg, out_deg, h_sp, agg_sp, deg_sp, rows_v, idx_v, ones_v, degz,
         sem_i, sem_g, sem_s, sem_d) = rest
    else:
        out_agg, h_sp, agg_sp, rows_v, idx_v, sem_i, sem_g, sem_s = rest
        deg_sp = ones_v = degz = sem_d = None

    c = lax.axis_index("c")
    s = lax.axis_index("s")

    zrows = agg_rows // NS
    base_z = s * zrows
    pltpu.sync_copy(hs.at[c, pl.ds(base_z, zrows)], h_sp.at[pl.ds(base_z, zrows)])
    _zero_rows(rows_v.at[0], CHUNK, HALF)
    off = 0
    while off < zrows:
        sz = min(CHUNK, zrows - off)
        pltpu.sync_copy(rows_v.at[0, pl.ds(0, sz)], agg_sp.at[pl.ds(base_z + off, sz)])
        off += sz
    if compute_deg:
        _zero_rows(degz, DEGZ_ROWS, LANES)
        off = 0
        while off < zrows:
            sz = min(DEGZ_ROWS, zrows - off)
            pltpu.sync_copy(degz.at[pl.ds(0, sz)], deg_sp.at[pl.ds(base_z + off, sz)])
            off += sz

        def fill_ones(i, _):
            ones_v[i, :] = jnp.ones((LANES,), jnp.float32)
            return 0

        lax.fori_loop(0, CHUNK, fill_ones, 0, unroll=False)
    plsc.subcore_barrier()

    base_row = s * rows_per_tile

    def fire_idx(t, b):
        pltpu.async_copy(idxm.at[base_row + t], idx_v.at[b], sem_i.at[b])

    def wait_idx(b):
        pltpu.make_async_copy(idxm.at[base_row], idx_v.at[b], sem_i.at[b]).wait()

    def fire_gather(b):
        pltpu.async_copy(h_sp.at[idx_v.at[b, 0]], rows_v.at[b], sem_g.at[b])

    def wait_gather(b):
        pltpu.make_async_copy(h_sp.at[idx_v.at[b, 0]], rows_v.at[b],
                              sem_g.at[b]).wait()

    def fire_scatter(b):
        pltpu.async_copy(rows_v.at[b], agg_sp.at[idx_v.at[b, 1]], sem_s.at[b],
                         add=True)
        if compute_deg:
            pltpu.async_copy(ones_v, deg_sp.at[idx_v.at[b, 1]], sem_d.at[b],
                             add=True)

    def wait_scatter(b):
        pltpu.make_async_copy(rows_v.at[b], agg_sp.at[idx_v.at[b, 1]],
                              sem_s.at[b]).wait()
        if compute_deg:
            pltpu.make_async_copy(ones_v, deg_sp.at[idx_v.at[b, 1]],
                                  sem_d.at[b]).wait()

    fire_idx(0, 0)
    fire_idx(1, 1)
    wait_idx(0)
    fire_gather(0)
    fire_idx(2, 2)
    wait_idx(1)
    fire_gather(1)
    wait_gather(0)
    fire_scatter(0)
    fire_idx(3, 3)
    wait_idx(2)
    fire_gather(2)
    wait_gather(1)
    fire_scatter(1)

    def step(g, _):
        for b in range(NBUF):
            t = g * NBUF + b
            wait_scatter(b)
            fire_idx(t, b)
            wait_idx((b + 3) % NBUF)
            fire_gather((b + 3) % NBUF)
            wait_gather((b + 2) % NBUF)
            fire_scatter((b + 2) % NBUF)
        return 0

    lax.fori_loop(1, rows_per_tile // NBUF, step, 0, unroll=False)

    wait_idx(3)
    fire_gather(3)
    wait_gather(2)
    fire_scatter(2)
    wait_gather(3)
    fire_scatter(3)
    for b in range(NBUF):
        wait_scatter(b)
    plsc.subcore_barrier()

    off = 0
    while off < zrows:
        sz = min(CHUNK, zrows - off)
        pltpu.sync_copy(agg_sp.at[pl.ds(base_z + off, sz)], rows_v.at[0, pl.ds(0, sz)])
        pltpu.sync_copy(rows_v.at[0, pl.ds(0, sz)], out_agg.at[c, pl.ds(base_z + off, sz)])
        off += sz
    if compute_deg:
        off = 0
        while off < zrows:
            sz = min(DEGZ_ROWS, zrows - off)
            pltpu.sync_copy(deg_sp.at[pl.ds(base_z + off, sz)], degz.at[pl.ds(0, sz)])
            pltpu.sync_copy(degz.at[pl.ds(0, sz)], out_deg.at[c, pl.ds(base_z + off, sz)])
            off += sz


def _segsum(hs, idxm, compute_deg):
    _, agg_rows, half = hs.shape
    assert half == HALF
    erows = idxm.shape[0]
    rows_per_tile = erows // NS
    assert agg_rows % (NS * 8) == 0 and rows_per_tile % NBUF == 0

    mesh = plsc.VectorSubcoreMesh(
        core_axis_name="c", subcore_axis_name="s", num_cores=NC, num_subcores=NS
    )
    out_type = [jax.ShapeDtypeStruct((NC, agg_rows, HALF), jnp.float32)]
    if compute_deg:
        out_type.append(jax.ShapeDtypeStruct((NC, agg_rows, LANES), jnp.float32))
        scratch = [
            pltpu.VMEM_SHARED((agg_rows, HALF), jnp.float32),
            pltpu.VMEM_SHARED((agg_rows, HALF), jnp.float32),
            pltpu.VMEM_SHARED((agg_rows, LANES), jnp.float32),
            pltpu.VMEM((NBUF, CHUNK, HALF), jnp.float32),
            pltpu.VMEM((NBUF, 2, CHUNK), jnp.int32),
            pltpu.VMEM((CHUNK, LANES), jnp.float32),
            pltpu.VMEM((DEGZ_ROWS, LANES), jnp.float32),
            pltpu.SemaphoreType.DMA((NBUF,)),
            pltpu.SemaphoreType.DMA((NBUF,)),
            pltpu.SemaphoreType.DMA((NBUF,)),
            pltpu.SemaphoreType.DMA((NBUF,)),
        ]
    else:
        scratch = [
            pltpu.VMEM_SHARED((agg_rows, HALF), jnp.float32),
            pltpu.VMEM_SHARED((agg_rows, HALF), jnp.float32),
            pltpu.VMEM((NBUF, CHUNK, HALF), jnp.float32),
            pltpu.VMEM((NBUF, 2, CHUNK), jnp.int32),
            pltpu.SemaphoreType.DMA((NBUF,)),
            pltpu.SemaphoreType.DMA((NBUF,)),
            pltpu.SemaphoreType.DMA((NBUF,)),
        ]

    body = functools.partial(_segsum_body, compute_deg, rows_per_tile, agg_rows)
    fn = pl.kernel(body, out_type=tuple(out_type), mesh=mesh,
                   scratch_types=tuple(scratch),
                   compiler_params=pltpu.CompilerParams(use_tc_tiling_on_sc=False))
    return fn(hs, idxm)


def _sage_block(n, hs, aggp, degp, wl, bl, wr, g, b):
    dp = degp[...]
    deg = (dp[0, :n, 0:1] + dp[1, :n, 0:1]) * 0.5
    inv = 1.0 / jnp.maximum(deg, 1.0)
    h = jnp.concatenate([hs[0, :n], hs[1, :n]], axis=1)
    agg = jnp.concatenate([aggp[0, :n], aggp[1, :n]], axis=1)
    mean = agg * inv
    lin = (
        jnp.dot(mean, wl[...], preferred_element_type=jnp.float32)
        + bl[...]
        + jnp.dot(h, wr[...], preferred_element_type=jnp.float32)
    )
    mu = jnp.mean(lin, axis=0, keepdims=True)
    xc = lin - mu
    var = jnp.mean(xc * xc, axis=0, keepdims=True)
    y = g[...] * xc * lax.rsqrt(var + 1e-5) + b[...]
    return jnp.maximum(y, 0.0)


def _layer_mid_body(n, hs, aggp, degp, wl, bl, wr, g, b, out_hs):
    npad = out_hs.shape[1]
    y = _sage_block(n, hs, aggp, degp, wl, bl, wr, g, b)
    ypad = jnp.pad(y, ((0, npad - n), (0, 0)))
    out_hs[0] = ypad[:, :HALF]
    out_hs[1] = ypad[:, HALF:]


def _layer_final_body(hs, aggp, degp, wl, bl, wr, g, b, wc, bc, out):
    n = out.shape[0]
    hr = _sage_block(n, hs, aggp, degp, wl, bl, wr, g, b)
    logits = jnp.dot(hr, wc[...], preferred_element_type=jnp.float32) + bc[...]
    m = jnp.max(logits, axis=1, keepdims=True)
    z = logits - m
    lse = jnp.log(jnp.sum(jnp.exp(z), axis=1, keepdims=True))
    out[...] = z - lse


def _layer_mid(n, hs, aggp, degp, wl, bl, wr, g, b):
    npad = aggp.shape[1]
    return pl.pallas_call(
        functools.partial(_layer_mid_body, n),
        out_shape=jax.ShapeDtypeStruct((NC, npad, HALF), jnp.float32),
    )(hs, aggp, degp, wl, bl.reshape(1, -1), wr, g.reshape(1, -1),
      b.reshape(1, -1))


def _layer_final(n, hs, aggp, degp, wl, bl, wr, g, b, wc, bc):
    ncls = wc.shape[1]
    return pl.pallas_call(
        _layer_final_body,
        out_shape=jax.ShapeDtypeStruct((n, ncls), jnp.float32),
    )(hs, aggp, degp, wl, bl.reshape(1, -1), wr, g.reshape(1, -1),
      b.reshape(1, -1), wc, bc.reshape(1, -1))


def kernel(x, edge_index, Wl0, bl0, Wr0, gamma0, beta0, Wl1, bl1, Wr1, gamma1, beta1, Wl2, bl2, Wr2, gamma2, beta2, Wc, bc):
    n = x.shape[0]
    npad = -(-(n + 1) // (NS * 8)) * (NS * 8)
    e = edge_index.shape[1]
    epad = -(-e // (CHUNK * NS * NBUF)) * (CHUNK * NS * NBUF)
    src = edge_index[0]
    dst = edge_index[1]
    if epad > e:
        pad = epad - e
        src = jnp.concatenate([src, jnp.zeros((pad,), jnp.int32)])
        dst = jnp.concatenate([dst, jnp.full((pad,), n, jnp.int32)])
    idxm = jnp.stack(
        [src.reshape(epad // CHUNK, CHUNK), dst.reshape(epad // CHUNK, CHUNK)],
        axis=1,
    )
    xpad = jnp.pad(x, ((0, npad - n), (0, 0)))
    hs0 = jnp.moveaxis(xpad.reshape(npad, NC, HALF), 1, 0)

    agg0, degp = _segsum(hs0, idxm, compute_deg=True)
    hs1 = _layer_mid(n, hs0, agg0, degp, Wl0, bl0, Wr0, gamma0, beta0)
    (agg1,) = _segsum(hs1, idxm, compute_deg=False)
    hs2 = _layer_mid(n, hs1, agg1, degp, Wl1, bl1, Wr1, gamma1, beta1)
    (agg2,) = _segsum(hs2, idxm, compute_deg=False)
    return _layer_final(n, hs2, agg2, degp, Wl2, bl2, Wr2, gamma2, beta2, Wc, bc)


# speedup vs baseline: 3.5200x; 1.3856x over previous
import functools

import jax
import jax.numpy as jnp
from jax import lax
from jax.experimental import pallas as pl
from jax.experimental.pallas import tpu as pltpu
from jax.experimental.pallas import tpu_sc as plsc

NC, NS, LANES = 2, 16, 16
HALF = 64
CHUNK = 128
NBUF = 4
DEGZ_ROWS = 128


def _zero_rows(ref, nrows, width, dtype=jnp.float32):
    lanes = LANES * (2 if dtype == jnp.bfloat16 else 1)

    def body(i, _):
        for j in range(width // lanes):
            ref[i, pl.ds(j * lanes, lanes)] = jnp.zeros((lanes,), dtype)
        return 0

    lax.fori_loop(0, nrows, body, 0, unroll=False)


def _segsum_body(compute_deg, rows_per_tile, agg_rows, hs, idxm, *rest):
    if compute_deg:
        (out_agg, out_deg, h_sp, agg_sp, deg_sp, rows_v, idx_v, ones_v, degz,
         sem_i, sem_g, sem_s, sem_d) = rest
    else:
        out_agg, h_sp, agg_sp, rows_v, idx_v, sem_i, sem_g, sem_s = rest
        deg_sp = ones_v = degz = sem_d = None

    c = lax.axis_index("c")
    s = lax.axis_index("s")

    zrows = agg_rows // NS
    base_z = s * zrows
    pltpu.sync_copy(hs.at[c, pl.ds(base_z, zrows)], h_sp.at[pl.ds(base_z, zrows)])
    _zero_rows(rows_v.at[0], CHUNK, HALF, jnp.bfloat16)
    off = 0
    while off < zrows:
        sz = min(CHUNK, zrows - off)
        pltpu.sync_copy(rows_v.at[0, pl.ds(0, sz)], agg_sp.at[pl.ds(base_z + off, sz)])
        off += sz
    if compute_deg:
        _zero_rows(degz, DEGZ_ROWS, LANES)
        off = 0
        while off < zrows:
            sz = min(DEGZ_ROWS, zrows - off)
            pltpu.sync_copy(degz.at[pl.ds(0, sz)], deg_sp.at[pl.ds(base_z + off, sz)])
            off += sz

        def fill_ones(i, _):
            ones_v[i, :] = jnp.ones((LANES,), jnp.float32)
            return 0

        lax.fori_loop(0, CHUNK, fill_ones, 0, unroll=False)
    plsc.subcore_barrier()

    base_row = s * rows_per_tile

    def fire_idx(t, b):
        pltpu.async_copy(idxm.at[base_row + t], idx_v.at[b], sem_i.at[b])

    def wait_idx(b):
        pltpu.make_async_copy(idxm.at[base_row], idx_v.at[b], sem_i.at[b]).wait()

    def fire_gather(b):
        pltpu.async_copy(h_sp.at[idx_v.at[b, 0]], rows_v.at[b], sem_g.at[b])

    def wait_gather(b):
        pltpu.make_async_copy(h_sp.at[idx_v.at[b, 0]], rows_v.at[b],
                              sem_g.at[b]).wait()

    def fire_scatter(b):
        pltpu.async_copy(rows_v.at[b], agg_sp.at[idx_v.at[b, 1]], sem_s.at[b],
                         add=True)
        if compute_deg:
            pltpu.async_copy(ones_v, deg_sp.at[idx_v.at[b, 1]], sem_d.at[b],
                             add=True)

    def wait_scatter(b):
        pltpu.make_async_copy(rows_v.at[b], agg_sp.at[idx_v.at[b, 1]],
                              sem_s.at[b]).wait()
        if compute_deg:
            pltpu.make_async_copy(ones_v, deg_sp.at[idx_v.at[b, 1]],
                                  sem_d.at[b]).wait()

    fire_idx(0, 0)
    fire_idx(1, 1)
    wait_idx(0)
    fire_gather(0)
    fire_idx(2, 2)
    wait_idx(1)
    fire_gather(1)
    wait_gather(0)
    fire_scatter(0)
    fire_idx(3, 3)
    wait_idx(2)
    fire_gather(2)
    wait_gather(1)
    fire_scatter(1)

    def step(g, _):
        for b in range(NBUF):
            t = g * NBUF + b
            wait_scatter(b)
            fire_idx(t, b)
            wait_idx((b + 3) % NBUF)
            fire_gather((b + 3) % NBUF)
            wait_gather((b + 2) % NBUF)
            fire_scatter((b + 2) % NBUF)
        return 0

    lax.fori_loop(1, rows_per_tile // NBUF, step, 0, unroll=False)

    wait_idx(3)
    fire_gather(3)
    wait_gather(2)
    fire_scatter(2)
    wait_gather(3)
    fire_scatter(3)
    for b in range(NBUF):
        wait_scatter(b)
    plsc.subcore_barrier()

    off = 0
    while off < zrows:
        sz = min(CHUNK, zrows - off)
        pltpu.sync_copy(agg_sp.at[pl.ds(base_z + off, sz)], rows_v.at[0, pl.ds(0, sz)])
        pltpu.sync_copy(rows_v.at[0, pl.ds(0, sz)], out_agg.at[c, pl.ds(base_z + off, sz)])
        off += sz
    if compute_deg:
        off = 0
        while off < zrows:
            sz = min(DEGZ_ROWS, zrows - off)
            pltpu.sync_copy(deg_sp.at[pl.ds(base_z + off, sz)], degz.at[pl.ds(0, sz)])
            pltpu.sync_copy(degz.at[pl.ds(0, sz)], out_deg.at[c, pl.ds(base_z + off, sz)])
            off += sz


def _segsum(hs, idxm, compute_deg):
    _, agg_rows, half = hs.shape
    assert half == HALF
    erows = idxm.shape[0]
    rows_per_tile = erows // NS
    assert agg_rows % (NS * 8) == 0 and rows_per_tile % NBUF == 0

    mesh = plsc.VectorSubcoreMesh(
        core_axis_name="c", subcore_axis_name="s", num_cores=NC, num_subcores=NS
    )
    out_type = [jax.ShapeDtypeStruct((NC, agg_rows, HALF), jnp.bfloat16)]
    if compute_deg:
        out_type.append(jax.ShapeDtypeStruct((NC, agg_rows, LANES), jnp.float32))
        scratch = [
            pltpu.VMEM_SHARED((agg_rows, HALF), jnp.bfloat16),
            pltpu.VMEM_SHARED((agg_rows, HALF), jnp.bfloat16),
            pltpu.VMEM_SHARED((agg_rows, LANES), jnp.float32),
            pltpu.VMEM((NBUF, CHUNK, HALF), jnp.bfloat16),
            pltpu.VMEM((NBUF, 2, CHUNK), jnp.int32),
            pltpu.VMEM((CHUNK, LANES), jnp.float32),
            pltpu.VMEM((DEGZ_ROWS, LANES), jnp.float32),
            pltpu.SemaphoreType.DMA((NBUF,)),
            pltpu.SemaphoreType.DMA((NBUF,)),
            pltpu.SemaphoreType.DMA((NBUF,)),
            pltpu.SemaphoreType.DMA((NBUF,)),
        ]
    else:
        scratch = [
            pltpu.VMEM_SHARED((agg_rows, HALF), jnp.bfloat16),
            pltpu.VMEM_SHARED((agg_rows, HALF), jnp.bfloat16),
            pltpu.VMEM((NBUF, CHUNK, HALF), jnp.bfloat16),
            pltpu.VMEM((NBUF, 2, CHUNK), jnp.int32),
            pltpu.SemaphoreType.DMA((NBUF,)),
            pltpu.SemaphoreType.DMA((NBUF,)),
            pltpu.SemaphoreType.DMA((NBUF,)),
        ]

    body = functools.partial(_segsum_body, compute_deg, rows_per_tile, agg_rows)
    fn = pl.kernel(body, out_type=tuple(out_type), mesh=mesh,
                   scratch_types=tuple(scratch),
                   compiler_params=pltpu.CompilerParams(use_tc_tiling_on_sc=False))
    return fn(hs, idxm)


def _sage_block(n, hs, aggp, degp, wl, bl, wr, g, b):
    dp = degp[...]
    deg = (dp[0, :n, 0:1] + dp[1, :n, 0:1]) * 0.5
    inv = 1.0 / jnp.maximum(deg, 1.0)
    h = jnp.concatenate([hs[0, :n], hs[1, :n]], axis=1).astype(jnp.float32)
    agg = jnp.concatenate([aggp[0, :n], aggp[1, :n]], axis=1).astype(jnp.float32)
    mean = agg * inv
    lin = (
        jnp.dot(mean, wl[...], preferred_element_type=jnp.float32)
        + bl[...]
        + jnp.dot(h, wr[...], preferred_element_type=jnp.float32)
    )
    mu = jnp.mean(lin, axis=0, keepdims=True)
    xc = lin - mu
    var = jnp.mean(xc * xc, axis=0, keepdims=True)
    y = g[...] * xc * lax.rsqrt(var + 1e-5) + b[...]
    return jnp.maximum(y, 0.0)


def _layer_mid_body(n, hs, aggp, degp, wl, bl, wr, g, b, out_hs):
    npad = out_hs.shape[1]
    y = _sage_block(n, hs, aggp, degp, wl, bl, wr, g, b)
    ypad = jnp.pad(y, ((0, npad - n), (0, 0))).astype(jnp.bfloat16)
    out_hs[0] = ypad[:, :HALF]
    out_hs[1] = ypad[:, HALF:]


def _layer_final_body(hs, aggp, degp, wl, bl, wr, g, b, wc, bc, out):
    n = out.shape[0]
    hr = _sage_block(n, hs, aggp, degp, wl, bl, wr, g, b)
    logits = jnp.dot(hr, wc[...], preferred_element_type=jnp.float32) + bc[...]
    m = jnp.max(logits, axis=1, keepdims=True)
    z = logits - m
    lse = jnp.log(jnp.sum(jnp.exp(z), axis=1, keepdims=True))
    out[...] = z - lse


def _layer_mid(n, hs, aggp, degp, wl, bl, wr, g, b):
    npad = aggp.shape[1]
    return pl.pallas_call(
        functools.partial(_layer_mid_body, n),
        out_shape=jax.ShapeDtypeStruct((NC, npad, HALF), jnp.bfloat16),
    )(hs, aggp, degp, wl, bl.reshape(1, -1), wr, g.reshape(1, -1),
      b.reshape(1, -1))


def _layer_final(n, hs, aggp, degp, wl, bl, wr, g, b, wc, bc):
    ncls = wc.shape[1]
    return pl.pallas_call(
        _layer_final_body,
        out_shape=jax.ShapeDtypeStruct((n, ncls), jnp.float32),
    )(hs, aggp, degp, wl, bl.reshape(1, -1), wr, g.reshape(1, -1),
      b.reshape(1, -1), wc, bc.reshape(1, -1))


def kernel(x, edge_index, Wl0, bl0, Wr0, gamma0, beta0, Wl1, bl1, Wr1, gamma1, beta1, Wl2, bl2, Wr2, gamma2, beta2, Wc, bc):
    n = x.shape[0]
    npad = -(-(n + 1) // (NS * 8)) * (NS * 8)
    e = edge_index.shape[1]
    epad = -(-e // (CHUNK * NS * NBUF)) * (CHUNK * NS * NBUF)
    src = edge_index[0]
    dst = edge_index[1]
    if epad > e:
        pad = epad - e
        src = jnp.concatenate([src, jnp.zeros((pad,), jnp.int32)])
        dst = jnp.concatenate([dst, jnp.full((pad,), n, jnp.int32)])
    idxm = jnp.stack(
        [src.reshape(epad // CHUNK, CHUNK), dst.reshape(epad // CHUNK, CHUNK)],
        axis=1,
    )
    xpad = jnp.pad(x, ((0, npad - n), (0, 0))).astype(jnp.bfloat16)
    hs0 = jnp.moveaxis(xpad.reshape(npad, NC, HALF), 1, 0)

    agg0, degp = _segsum(hs0, idxm, compute_deg=True)
    hs1 = _layer_mid(n, hs0, agg0, degp, Wl0, bl0, Wr0, gamma0, beta0)
    (agg1,) = _segsum(hs1, idxm, compute_deg=False)
    hs2 = _layer_mid(n, hs1, agg1, degp, Wl1, bl1, Wr1, gamma1, beta1)
    (agg2,) = _segsum(hs2, idxm, compute_deg=False)
    return _layer_final(n, hs2, agg2, degp, Wl2, bl2, Wr2, gamma2, beta2, Wc, bc)


# speedup vs baseline: 3.7249x; 1.0582x over previous
import functools

import jax
import jax.numpy as jnp
from jax import lax
from jax.experimental import pallas as pl
from jax.experimental.pallas import tpu as pltpu
from jax.experimental.pallas import tpu_sc as plsc

NC, NS, LANES = 2, 16, 16
HALF = 64
CHUNK = 128
NBUF = 4
DEGZ_ROWS = 128


def _zero_rows(ref, nrows, width, dtype=jnp.float32):
    lanes = LANES * (2 if dtype == jnp.bfloat16 else 1)

    def body(i, _):
        for j in range(width // lanes):
            ref[i, pl.ds(j * lanes, lanes)] = jnp.zeros((lanes,), dtype)
        return 0

    lax.fori_loop(0, nrows, body, 0, unroll=False)


def _segsum_body(compute_deg, rows_per_tile, agg_rows, hs, idxm, *rest):
    if compute_deg:
        (out_agg, out_deg, h_sp, agg_sp, deg_sp, rows_v, idx_v, ones_v, degz,
         sem_i, sem_g, sem_s, sem_d) = rest
    else:
        out_agg, h_sp, agg_sp, rows_v, idx_v, sem_i, sem_g, sem_s = rest
        deg_sp = ones_v = degz = sem_d = None

    c = lax.axis_index("c")
    s = lax.axis_index("s")

    zrows = agg_rows // NS
    base_z = s * zrows
    pltpu.sync_copy(hs.at[c, pl.ds(base_z, zrows)], h_sp.at[pl.ds(base_z, zrows)])
    _zero_rows(rows_v.at[0], CHUNK, HALF, jnp.bfloat16)
    off = 0
    while off < zrows:
        sz = min(CHUNK, zrows - off)
        pltpu.sync_copy(rows_v.at[0, pl.ds(0, sz)], agg_sp.at[pl.ds(base_z + off, sz)])
        off += sz
    if compute_deg:
        _zero_rows(degz, DEGZ_ROWS, LANES)
        off = 0
        while off < zrows:
            sz = min(DEGZ_ROWS, zrows - off)
            pltpu.sync_copy(degz.at[pl.ds(0, sz)], deg_sp.at[pl.ds(base_z + off, sz)])
            off += sz

        def fill_ones(i, _):
            ones_v[i, :] = jnp.ones((LANES,), jnp.float32)
            return 0

        lax.fori_loop(0, CHUNK, fill_ones, 0, unroll=False)
    plsc.subcore_barrier()

    base_row = s * rows_per_tile

    def fire_idx(t, b):
        pltpu.async_copy(idxm.at[base_row + t], idx_v.at[b], sem_i.at[b])

    def wait_idx(b):
        pltpu.make_async_copy(idxm.at[base_row], idx_v.at[b], sem_i.at[b]).wait()

    def fire_gather(b):
        pltpu.async_copy(h_sp.at[idx_v.at[b, 0]], rows_v.at[b], sem_g.at[b])

    def wait_gather(b):
        pltpu.make_async_copy(h_sp.at[idx_v.at[b, 0]], rows_v.at[b],
                              sem_g.at[b]).wait()

    half_t = rows_per_tile // 2

    def deg_mine(t):
        return (c == 0) == (t < half_t)

    def fire_scatter(b, t):
        pltpu.async_copy(rows_v.at[b], agg_sp.at[idx_v.at[b, 1]], sem_s.at[b],
                         add=True)
        if compute_deg:
            @pl.when(deg_mine(t))
            def _():
                pltpu.async_copy(ones_v, deg_sp.at[idx_v.at[b, 1]], sem_d.at[b],
                                 add=True)

    def wait_scatter(b, t):
        pltpu.make_async_copy(rows_v.at[b], agg_sp.at[idx_v.at[b, 1]],
                              sem_s.at[b]).wait()
        if compute_deg:
            @pl.when(deg_mine(t))
            def _():
                pltpu.make_async_copy(ones_v, deg_sp.at[idx_v.at[b, 1]],
                                      sem_d.at[b]).wait()

    fire_idx(0, 0)
    fire_idx(1, 1)
    wait_idx(0)
    fire_gather(0)
    fire_idx(2, 2)
    wait_idx(1)
    fire_gather(1)
    wait_gather(0)
    fire_scatter(0, 0)
    fire_idx(3, 3)
    wait_idx(2)
    fire_gather(2)
    wait_gather(1)
    fire_scatter(1, 1)

    def step(g, _):
        for b in range(NBUF):
            t = g * NBUF + b
            wait_scatter(b, t - 4)
            fire_idx(t, b)
            wait_idx((b + 3) % NBUF)
            fire_gather((b + 3) % NBUF)
            wait_gather((b + 2) % NBUF)
            fire_scatter((b + 2) % NBUF, t - 2)
        return 0

    lax.fori_loop(1, rows_per_tile // NBUF, step, 0, unroll=False)

    last = rows_per_tile
    wait_idx(3)
    fire_gather(3)
    wait_gather(2)
    fire_scatter(2, last - 2)
    wait_gather(3)
    fire_scatter(3, last - 1)
    for b in range(NBUF):
        wait_scatter(b, last - 4 + b)
    plsc.subcore_barrier()

    off = 0
    while off < zrows:
        sz = min(CHUNK, zrows - off)
        pltpu.sync_copy(agg_sp.at[pl.ds(base_z + off, sz)], rows_v.at[0, pl.ds(0, sz)])
        pltpu.sync_copy(rows_v.at[0, pl.ds(0, sz)], out_agg.at[c, pl.ds(base_z + off, sz)])
        off += sz
    if compute_deg:
        off = 0
        while off < zrows:
            sz = min(DEGZ_ROWS, zrows - off)
            pltpu.sync_copy(deg_sp.at[pl.ds(base_z + off, sz)], degz.at[pl.ds(0, sz)])
            pltpu.sync_copy(degz.at[pl.ds(0, sz)], out_deg.at[c, pl.ds(base_z + off, sz)])
            off += sz


def _segsum(hs, idxm, compute_deg):
    _, agg_rows, half = hs.shape
    assert half == HALF
    erows = idxm.shape[0]
    rows_per_tile = erows // NS
    assert agg_rows % (NS * 8) == 0 and rows_per_tile % NBUF == 0

    mesh = plsc.VectorSubcoreMesh(
        core_axis_name="c", subcore_axis_name="s", num_cores=NC, num_subcores=NS
    )
    out_type = [jax.ShapeDtypeStruct((NC, agg_rows, HALF), jnp.bfloat16)]
    if compute_deg:
        out_type.append(jax.ShapeDtypeStruct((NC, agg_rows, LANES), jnp.float32))
        scratch = [
            pltpu.VMEM_SHARED((agg_rows, HALF), jnp.bfloat16),
            pltpu.VMEM_SHARED((agg_rows, HALF), jnp.bfloat16),
            pltpu.VMEM_SHARED((agg_rows, LANES), jnp.float32),
            pltpu.VMEM((NBUF, CHUNK, HALF), jnp.bfloat16),
            pltpu.VMEM((NBUF, 2, CHUNK), jnp.int32),
            pltpu.VMEM((CHUNK, LANES), jnp.float32),
            pltpu.VMEM((DEGZ_ROWS, LANES), jnp.float32),
            pltpu.SemaphoreType.DMA((NBUF,)),
            pltpu.SemaphoreType.DMA((NBUF,)),
            pltpu.SemaphoreType.DMA((NBUF,)),
            pltpu.SemaphoreType.DMA((NBUF,)),
        ]
    else:
        scratch = [
            pltpu.VMEM_SHARED((agg_rows, HALF), jnp.bfloat16),
            pltpu.VMEM_SHARED((agg_rows, HALF), jnp.bfloat16),
            pltpu.VMEM((NBUF, CHUNK, HALF), jnp.bfloat16),
            pltpu.VMEM((NBUF, 2, CHUNK), jnp.int32),
            pltpu.SemaphoreType.DMA((NBUF,)),
            pltpu.SemaphoreType.DMA((NBUF,)),
            pltpu.SemaphoreType.DMA((NBUF,)),
        ]

    body = functools.partial(_segsum_body, compute_deg, rows_per_tile, agg_rows)
    fn = pl.kernel(body, out_type=tuple(out_type), mesh=mesh,
                   scratch_types=tuple(scratch),
                   compiler_params=pltpu.CompilerParams(use_tc_tiling_on_sc=False))
    return fn(hs, idxm)


def _sage_block(n, hs, aggp, degp, wl, bl, wr, g, b):
    dp = degp[...]
    deg = dp[0, :n, 0:1] + dp[1, :n, 0:1]
    inv = 1.0 / jnp.maximum(deg, 1.0)
    h = jnp.concatenate([hs[0, :n], hs[1, :n]], axis=1).astype(jnp.float32)
    agg = jnp.concatenate([aggp[0, :n], aggp[1, :n]], axis=1).astype(jnp.float32)
    mean = agg * inv
    lin = (
        jnp.dot(mean, wl[...], preferred_element_type=jnp.float32)
        + bl[...]
        + jnp.dot(h, wr[...], preferred_element_type=jnp.float32)
    )
    mu = jnp.mean(lin, axis=0, keepdims=True)
    xc = lin - mu
    var = jnp.mean(xc * xc, axis=0, keepdims=True)
    y = g[...] * xc * lax.rsqrt(var + 1e-5) + b[...]
    return jnp.maximum(y, 0.0)


def _prep_body(n, x, out_hs):
    xr = x[...].astype(jnp.bfloat16)
    out_hs[0, :n] = xr[:, :HALF]
    out_hs[1, :n] = xr[:, HALF:]


def _layer_mid_body(n, hs, aggp, degp, wl, bl, wr, g, b, out_hs):
    npad = out_hs.shape[1]
    y = _sage_block(n, hs, aggp, degp, wl, bl, wr, g, b)
    ypad = jnp.pad(y, ((0, npad - n), (0, 0))).astype(jnp.bfloat16)
    out_hs[0] = ypad[:, :HALF]
    out_hs[1] = ypad[:, HALF:]


def _layer_final_body(hs, aggp, degp, wl, bl, wr, g, b, wc, bc, out):
    n = out.shape[0]
    hr = _sage_block(n, hs, aggp, degp, wl, bl, wr, g, b)
    logits = jnp.dot(hr, wc[...], preferred_element_type=jnp.float32) + bc[...]
    m = jnp.max(logits, axis=1, keepdims=True)
    z = logits - m
    lse = jnp.log(jnp.sum(jnp.exp(z), axis=1, keepdims=True))
    out[...] = z - lse


def _layer_mid(n, hs, aggp, degp, wl, bl, wr, g, b):
    npad = aggp.shape[1]
    return pl.pallas_call(
        functools.partial(_layer_mid_body, n),
        out_shape=jax.ShapeDtypeStruct((NC, npad, HALF), jnp.bfloat16),
    )(hs, aggp, degp, wl, bl.reshape(1, -1), wr, g.reshape(1, -1),
      b.reshape(1, -1))


def _layer_final(n, hs, aggp, degp, wl, bl, wr, g, b, wc, bc):
    ncls = wc.shape[1]
    return pl.pallas_call(
        _layer_final_body,
        out_shape=jax.ShapeDtypeStruct((n, ncls), jnp.float32),
    )(hs, aggp, degp, wl, bl.reshape(1, -1), wr, g.reshape(1, -1),
      b.reshape(1, -1), wc, bc.reshape(1, -1))


def kernel(x, edge_index, Wl0, bl0, Wr0, gamma0, beta0, Wl1, bl1, Wr1, gamma1, beta1, Wl2, bl2, Wr2, gamma2, beta2, Wc, bc):
    n = x.shape[0]
    npad = -(-(n + 1) // (NS * 8)) * (NS * 8)
    e = edge_index.shape[1]
    epad = -(-e // (CHUNK * NS * NBUF)) * (CHUNK * NS * NBUF)
    src = edge_index[0]
    dst = edge_index[1]
    if epad > e:
        pad = epad - e
        src = jnp.concatenate([src, jnp.zeros((pad,), jnp.int32)])
        dst = jnp.concatenate([dst, jnp.full((pad,), n, jnp.int32)])
    idxm = jnp.stack(
        [src.reshape(epad // CHUNK, CHUNK), dst.reshape(epad // CHUNK, CHUNK)],
        axis=1,
    )
    hs0 = pl.pallas_call(
        functools.partial(_prep_body, n),
        out_shape=jax.ShapeDtypeStruct((NC, npad, HALF), jnp.bfloat16),
    )(x)

    agg0, degp = _segsum(hs0, idxm, compute_deg=True)
    hs1 = _layer_mid(n, hs0, agg0, degp, Wl0, bl0, Wr0, gamma0, beta0)
    (agg1,) = _segsum(hs1, idxm, compute_deg=False)
    hs2 = _layer_mid(n, hs1, agg1, degp, Wl1, bl1, Wr1, gamma1, beta1)
    (agg2,) = _segsum(hs2, idxm, compute_deg=False)
    return _layer_final(n, hs2, agg2, degp, Wl2, bl2, Wr2, gamma2, beta2, Wc, bc)


# speedup vs baseline: 3.8230x; 1.0263x over previous
import functools

import jax
import jax.numpy as jnp
from jax import lax
from jax.experimental import pallas as pl
from jax.experimental.pallas import tpu as pltpu
from jax.experimental.pallas import tpu_sc as plsc

NC, NS, LANES = 2, 16, 16
HALF = 64
CHUNK = 128
NBUF = 4
DEGZ_ROWS = 128


def _zero_rows(ref, nrows, width, dtype=jnp.float32):
    lanes = LANES * (2 if dtype == jnp.bfloat16 else 1)

    def body(i, _):
        for j in range(width // lanes):
            ref[i, pl.ds(j * lanes, lanes)] = jnp.zeros((lanes,), dtype)
        return 0

    lax.fori_loop(0, nrows, body, 0, unroll=False)


def _segsum_body(compute_deg, rows_per_tile, agg_rows, hs, idxm, *rest):
    if compute_deg:
        (out_agg, out_deg, h_sp, agg_sp, deg_sp, rows_v, idx_v, ones_v, degz,
         sem_i, sem_g, sem_s, sem_d) = rest
    else:
        out_agg, h_sp, agg_sp, rows_v, idx_v, sem_i, sem_g, sem_s = rest
        deg_sp = ones_v = degz = sem_d = None

    c = lax.axis_index("c")
    s = lax.axis_index("s")

    zrows = agg_rows // NS
    base_z = s * zrows
    pltpu.sync_copy(hs.at[c, pl.ds(base_z, zrows)], h_sp.at[pl.ds(base_z, zrows)])
    _zero_rows(rows_v.at[0], CHUNK, HALF, jnp.bfloat16)
    off = 0
    while off < zrows:
        sz = min(CHUNK, zrows - off)
        pltpu.sync_copy(rows_v.at[0, pl.ds(0, sz)], agg_sp.at[pl.ds(base_z + off, sz)])
        off += sz
    if compute_deg:
        _zero_rows(degz, DEGZ_ROWS, LANES)
        off = 0
        while off < zrows:
            sz = min(DEGZ_ROWS, zrows - off)
            pltpu.sync_copy(degz.at[pl.ds(0, sz)], deg_sp.at[pl.ds(base_z + off, sz)])
            off += sz

        def fill_ones(i, _):
            ones_v[i, :] = jnp.ones((LANES,), jnp.float32)
            return 0

        lax.fori_loop(0, CHUNK, fill_ones, 0, unroll=False)
    plsc.subcore_barrier()

    base_row = s * rows_per_tile

    def fire_idx(t, b):
        pltpu.async_copy(idxm.at[base_row + t], idx_v.at[b], sem_i.at[b])

    def wait_idx(b):
        pltpu.make_async_copy(idxm.at[base_row], idx_v.at[b], sem_i.at[b]).wait()

    def fire_gather(b):
        pltpu.async_copy(h_sp.at[idx_v.at[b, 0]], rows_v.at[b], sem_g.at[b])

    def wait_gather(b):
        pltpu.make_async_copy(h_sp.at[idx_v.at[b, 0]], rows_v.at[b],
                              sem_g.at[b]).wait()

    half_t = rows_per_tile // 2

    def deg_mine(t):
        return (c == 0) == (t < half_t)

    def fire_scatter(b, t):
        pltpu.async_copy(rows_v.at[b], agg_sp.at[idx_v.at[b, 1]], sem_s.at[b],
                         add=True)
        if compute_deg:
            @pl.when(deg_mine(t))
            def _():
                pltpu.async_copy(ones_v, deg_sp.at[idx_v.at[b, 1]], sem_d.at[b],
                                 add=True)

    def wait_scatter(b, t):
        pltpu.make_async_copy(rows_v.at[b], agg_sp.at[idx_v.at[b, 1]],
                              sem_s.at[b]).wait()
        if compute_deg:
            @pl.when(deg_mine(t))
            def _():
                pltpu.make_async_copy(ones_v, deg_sp.at[idx_v.at[b, 1]],
                                      sem_d.at[b]).wait()

    fire_idx(0, 0)
    fire_idx(1, 1)
    wait_idx(0)
    fire_gather(0)
    fire_idx(2, 2)
    wait_idx(1)
    fire_gather(1)
    wait_gather(0)
    fire_scatter(0, 0)
    fire_idx(3, 3)
    wait_idx(2)
    fire_gather(2)
    wait_gather(1)
    fire_scatter(1, 1)

    def step(g, _):
        for b in range(NBUF):
            t = g * NBUF + b
            wait_scatter(b, t - 4)
            fire_idx(t, b)
            wait_idx((b + 3) % NBUF)
            fire_gather((b + 3) % NBUF)
            wait_gather((b + 2) % NBUF)
            fire_scatter((b + 2) % NBUF, t - 2)
        return 0

    lax.fori_loop(1, rows_per_tile // NBUF, step, 0, unroll=False)

    last = rows_per_tile
    wait_idx(3)
    fire_gather(3)
    wait_gather(2)
    fire_scatter(2, last - 2)
    wait_gather(3)
    fire_scatter(3, last - 1)
    for b in range(NBUF):
        wait_scatter(b, last - 4 + b)
    plsc.subcore_barrier()

    off = 0
    while off < zrows:
        sz = min(CHUNK, zrows - off)
        pltpu.sync_copy(agg_sp.at[pl.ds(base_z + off, sz)], rows_v.at[0, pl.ds(0, sz)])
        pltpu.sync_copy(rows_v.at[0, pl.ds(0, sz)], out_agg.at[c, pl.ds(base_z + off, sz)])
        off += sz
    if compute_deg:
        off = 0
        while off < zrows:
            sz = min(DEGZ_ROWS, zrows - off)
            pltpu.sync_copy(deg_sp.at[pl.ds(base_z + off, sz)], degz.at[pl.ds(0, sz)])
            pltpu.sync_copy(degz.at[pl.ds(0, sz)], out_deg.at[c, pl.ds(base_z + off, sz)])
            off += sz


def _segsum(hs, idxm, compute_deg):
    _, agg_rows, half = hs.shape
    assert half == HALF
    erows = idxm.shape[0]
    rows_per_tile = erows // NS
    assert agg_rows % (NS * 8) == 0 and rows_per_tile % NBUF == 0

    mesh = plsc.VectorSubcoreMesh(
        core_axis_name="c", subcore_axis_name="s", num_cores=NC, num_subcores=NS
    )
    out_type = [jax.ShapeDtypeStruct((NC, agg_rows, HALF), jnp.bfloat16)]
    if compute_deg:
        out_type.append(jax.ShapeDtypeStruct((NC, agg_rows, LANES), jnp.float32))
        scratch = [
            pltpu.VMEM_SHARED((agg_rows, HALF), jnp.bfloat16),
            pltpu.VMEM_SHARED((agg_rows, HALF), jnp.bfloat16),
            pltpu.VMEM_SHARED((agg_rows, LANES), jnp.float32),
            pltpu.VMEM((NBUF, CHUNK, HALF), jnp.bfloat16),
            pltpu.VMEM((NBUF, 2, CHUNK), jnp.int32),
            pltpu.VMEM((CHUNK, LANES), jnp.float32),
            pltpu.VMEM((DEGZ_ROWS, LANES), jnp.float32),
            pltpu.SemaphoreType.DMA((NBUF,)),
            pltpu.SemaphoreType.DMA((NBUF,)),
            pltpu.SemaphoreType.DMA((NBUF,)),
            pltpu.SemaphoreType.DMA((NBUF,)),
        ]
    else:
        scratch = [
            pltpu.VMEM_SHARED((agg_rows, HALF), jnp.bfloat16),
            pltpu.VMEM_SHARED((agg_rows, HALF), jnp.bfloat16),
            pltpu.VMEM((NBUF, CHUNK, HALF), jnp.bfloat16),
            pltpu.VMEM((NBUF, 2, CHUNK), jnp.int32),
            pltpu.SemaphoreType.DMA((NBUF,)),
            pltpu.SemaphoreType.DMA((NBUF,)),
            pltpu.SemaphoreType.DMA((NBUF,)),
        ]

    body = functools.partial(_segsum_body, compute_deg, rows_per_tile, agg_rows)
    fn = pl.kernel(body, out_type=tuple(out_type), mesh=mesh,
                   scratch_types=tuple(scratch),
                   compiler_params=pltpu.CompilerParams(use_tc_tiling_on_sc=False))
    return fn(hs, idxm)


def _sage_block(n, hs, aggp, degp, wl, bl, wr, g, b):
    dp = degp[...]
    deg = dp[0, :n, 0:1] + dp[1, :n, 0:1]
    inv = 1.0 / jnp.maximum(deg, 1.0)
    h = jnp.concatenate([hs[0, :n], hs[1, :n]], axis=1).astype(jnp.float32)
    agg = jnp.concatenate([aggp[0, :n], aggp[1, :n]], axis=1).astype(jnp.float32)
    mean = agg * inv
    lin = (
        jnp.dot(mean, wl[...], preferred_element_type=jnp.float32)
        + bl[...]
        + jnp.dot(h, wr[...], preferred_element_type=jnp.float32)
    )
    mu = jnp.mean(lin, axis=0, keepdims=True)
    xc = lin - mu
    var = jnp.mean(xc * xc, axis=0, keepdims=True)
    y = g[...] * xc * lax.rsqrt(var + 1e-5) + b[...]
    return jnp.maximum(y, 0.0)


def _prep_body(n, x, ei, out_hs, out_idx):
    xr = x[...].astype(jnp.bfloat16)
    out_hs[0, :n] = xr[:, :HALF]
    out_hs[1, :n] = xr[:, HALF:]
    er = ei[...]
    erows = er.shape[1]
    pad_rows = out_idx.shape[0] - erows
    out_idx[:erows, 0] = er[0]
    out_idx[:erows, 1] = er[1]
    out_idx[erows:, 0] = jnp.zeros((pad_rows, CHUNK), jnp.int32)
    out_idx[erows:, 1] = jnp.full((pad_rows, CHUNK), n, jnp.int32)


def _layer_mid_body(n, hs, aggp, degp, wl, bl, wr, g, b, out_hs):
    npad = out_hs.shape[1]
    y = _sage_block(n, hs, aggp, degp, wl, bl, wr, g, b)
    ypad = jnp.pad(y, ((0, npad - n), (0, 0))).astype(jnp.bfloat16)
    out_hs[0] = ypad[:, :HALF]
    out_hs[1] = ypad[:, HALF:]


def _layer_final_body(hs, aggp, degp, wl, bl, wr, g, b, wc, bc, out):
    n = out.shape[0]
    hr = _sage_block(n, hs, aggp, degp, wl, bl, wr, g, b)
    logits = jnp.dot(hr, wc[...], preferred_element_type=jnp.float32) + bc[...]
    m = jnp.max(logits, axis=1, keepdims=True)
    z = logits - m
    lse = jnp.log(jnp.sum(jnp.exp(z), axis=1, keepdims=True))
    out[...] = z - lse


def _layer_mid(n, hs, aggp, degp, wl, bl, wr, g, b):
    npad = aggp.shape[1]
    return pl.pallas_call(
        functools.partial(_layer_mid_body, n),
        out_shape=jax.ShapeDtypeStruct((NC, npad, HALF), jnp.bfloat16),
    )(hs, aggp, degp, wl, bl.reshape(1, -1), wr, g.reshape(1, -1),
      b.reshape(1, -1))


def _layer_final(n, hs, aggp, degp, wl, bl, wr, g, b, wc, bc):
    ncls = wc.shape[1]
    return pl.pallas_call(
        _layer_final_body,
        out_shape=jax.ShapeDtypeStruct((n, ncls), jnp.float32),
    )(hs, aggp, degp, wl, bl.reshape(1, -1), wr, g.reshape(1, -1),
      b.reshape(1, -1), wc, bc.reshape(1, -1))


def kernel(x, edge_index, Wl0, bl0, Wr0, gamma0, beta0, Wl1, bl1, Wr1, gamma1, beta1, Wl2, bl2, Wr2, gamma2, beta2, Wc, bc):
    n = x.shape[0]
    npad = -(-(n + 1) // (NS * 8)) * (NS * 8)
    e = edge_index.shape[1]
    assert e % CHUNK == 0
    epad = -(-e // (CHUNK * NS * NBUF)) * (CHUNK * NS * NBUF)
    hs0, idxm = pl.pallas_call(
        functools.partial(_prep_body, n),
        out_shape=(
            jax.ShapeDtypeStruct((NC, npad, HALF), jnp.bfloat16),
            jax.ShapeDtypeStruct((epad // CHUNK, 2, CHUNK), jnp.int32),
        ),
    )(x, edge_index.reshape(2, e // CHUNK, CHUNK))

    agg0, degp = _segsum(hs0, idxm, compute_deg=True)
    hs1 = _layer_mid(n, hs0, agg0, degp, Wl0, bl0, Wr0, gamma0, beta0)
    (agg1,) = _segsum(hs1, idxm, compute_deg=False)
    hs2 = _layer_mid(n, hs1, agg1, degp, Wl1, bl1, Wr1, gamma1, beta1)
    (agg2,) = _segsum(hs2, idxm, compute_deg=False)
    return _layer_final(n, hs2, agg2, degp, Wl2, bl2, Wr2, gamma2, beta2, Wc, bc)
